# Initial kernel scaffold; baseline (speedup 1.0000x reference)
import functools

import jax
import jax.numpy as jnp
from jax import lax
from jax.experimental import pallas as pl
from jax.experimental.pallas import tpu as pltpu
from jax.experimental.pallas import tpu_sc as plsc

FILTERS = 16
KSIZE = 3
GAMMA = 1.0

B, H, W, C = 4, 32, 32, 16
D = C * KSIZE * KSIZE
S2 = 2 * D
NW = 32
PIX = B * H * W
PPW = PIX // NW
ROWS_PER_W = PPW // W
NEWTON_ITERS = 16


def _sc_spike_conv(xpad, wh):
    mesh = plsc.VectorSubcoreMesh(core_axis_name="c", subcore_axis_name="s")

    @functools.partial(
        pl.kernel,
        out_type=jax.ShapeDtypeStruct((PIX, FILTERS), jnp.float32),
        mesh=mesh,
        scratch_types=[
            pltpu.VMEM((ROWS_PER_W + 2, W + 2, C), jnp.float32),
            pltpu.VMEM((D, FILTERS), jnp.float32),
            pltpu.VMEM((D, FILTERS), jnp.float32),
            pltpu.VMEM((D, FILTERS), jnp.float32),
            pltpu.VMEM((PPW, FILTERS), jnp.float32),
        ],
    )
    def k(xpad_hbm, wh_hbm, out_hbm, slab_v, wh_v, ma_v, mb_v, out_v):
        wid = lax.axis_index("s") * 2 + lax.axis_index("c")
        img = wid // (H // ROWS_PER_W)
        row0 = (wid % (H // ROWS_PER_W)) * ROWS_PER_W
        pltpu.sync_copy(xpad_hbm.at[img, pl.ds(row0, ROWS_PER_W + 2)], slab_v)
        pltpu.sync_copy(wh_hbm, wh_v)

        phi0 = jnp.full((FILTERS,), GAMMA / S2, dtype=jnp.float32)
        zero = jnp.zeros((FILTERS,), dtype=jnp.float32)

        def pixel_body(p, _):
            r = p // W
            col = p - r * W

            for dij in range(KSIZE * KSIZE):
                di, dj = dij // KSIZE, dij % KSIZE

                def build_c(c, _, di=di, dj=dj, dij=dij):
                    x = slab_v[r + di, col + dj, c]
                    wv = wh_v[dij * C + c]
                    ma_v[dij * C + c] = jnp.abs(x + wv)
                    mb_v[dij * C + c] = jnp.abs(x - wv)
                    return 0

                lax.fori_loop(0, C, build_c, 0, unroll=4)

            def newton(_, phis):
                pa, pb = phis

                def dloop(d, carry):
                    ga1, ga2, ca, gb1, gb2, cb = carry
                    ma = ma_v[d]
                    mb = mb_v[d]
                    s1a = pa + ma
                    s2a = pa - ma
                    s1b = pb + mb
                    s2b = pb - mb
                    ga1 = ga1 + jnp.maximum(s1a, 0.0)
                    ga2 = ga2 + jnp.maximum(s2a, 0.0)
                    gb1 = gb1 + jnp.maximum(s1b, 0.0)
                    gb2 = gb2 + jnp.maximum(s2b, 0.0)
                    ca = ca + jnp.where(s1a > 0.0, 1.0, 0.0) \
                            + jnp.where(s2a > 0.0, 1.0, 0.0)
                    cb = cb + jnp.where(s1b > 0.0, 1.0, 0.0) \
                            + jnp.where(s2b > 0.0, 1.0, 0.0)
                    return ga1, ga2, ca, gb1, gb2, cb

                ga1, ga2, ca, gb1, gb2, cb = lax.fori_loop(
                    0, D, dloop, (zero, zero, zero, zero, zero, zero), unroll=2)
                ca = jnp.maximum(ca, 1.0)
                cb = jnp.maximum(cb, 1.0)
                pa = pa - (ga1 + ga2 - GAMMA) / ca
                pb = pb - (gb1 + gb2 - GAMMA) / cb
                return pa, pb

            pa, pb = lax.fori_loop(0, NEWTON_ITERS, newton, (phi0, phi0))
            out_v[p] = jnp.maximum(pa - pb, 0.0)
            return 0

        lax.fori_loop(0, PPW, pixel_body, 0)
        pltpu.sync_copy(out_v, out_hbm.at[pl.ds(wid * PPW, PPW)])

    return k(xpad, wh)


def kernel(inputs, kernel):
    xpad = jnp.pad(inputs, ((0, 0), (1, 1), (1, 1), (0, 0)))
    wh = kernel * 0.5
    out = _sc_spike_conv(xpad, wh)
    return out.reshape(B, H, W, FILTERS)


# speedup vs baseline: 20.6647x; 20.6647x over previous
import functools

import jax
import jax.numpy as jnp
from jax import lax
from jax.experimental import pallas as pl
from jax.experimental.pallas import tpu as pltpu
from jax.experimental.pallas import tpu_sc as plsc

FILTERS = 16
KSIZE = 3
GAMMA = 1.0

B, H, W, C = 4, 32, 32, 16
D = C * KSIZE * KSIZE
S2 = 2 * D
NW = 32
PIX = B * H * W
PPW = PIX // NW
ROWS_PER_W = PPW // W
NEWTON_ITERS = 16


def _sc_spike_conv(xb, wh):
    mesh = plsc.VectorSubcoreMesh(core_axis_name="c", subcore_axis_name="s")

    @functools.partial(
        pl.kernel,
        out_type=jax.ShapeDtypeStruct((PIX, FILTERS), jnp.float32),
        mesh=mesh,
        compiler_params=pltpu.CompilerParams(use_tc_tiling_on_sc=False),
        scratch_types=[
            pltpu.VMEM((ROWS_PER_W + 2, W + 2, C, FILTERS), jnp.float32),
            pltpu.VMEM((D, FILTERS), jnp.float32),
            pltpu.VMEM((D, FILTERS), jnp.float32),
            pltpu.VMEM((D, FILTERS), jnp.float32),
            pltpu.VMEM((PPW, FILTERS), jnp.float32),
        ],
    )
    def k(xb_hbm, wh_hbm, out_hbm, slab_v, wh_v, ma_v, mb_v, out_v):
        wid = lax.axis_index("s") * 2 + lax.axis_index("c")
        img = wid // (H // ROWS_PER_W)
        row0 = (wid % (H // ROWS_PER_W)) * ROWS_PER_W
        pltpu.sync_copy(xb_hbm.at[img, pl.ds(row0, ROWS_PER_W + 2)], slab_v)
        pltpu.sync_copy(wh_hbm, wh_v)

        phi0 = jnp.full((FILTERS,), GAMMA / S2, dtype=jnp.float32)
        zero = jnp.zeros((FILTERS,), dtype=jnp.float32)

        def pixel_body(p, _):
            r = p // W
            col = p - r * W

            for dij in range(KSIZE * KSIZE):
                di, dj = dij // KSIZE, dij % KSIZE

                def build_c(c, _, di=di, dj=dj, dij=dij):
                    x = slab_v[r + di, col + dj, c]
                    wv = wh_v[dij * C + c]
                    ma_v[dij * C + c] = jnp.abs(x + wv)
                    mb_v[dij * C + c] = jnp.abs(x - wv)
                    return 0

                lax.fori_loop(0, C, build_c, 0, unroll=4)

            def newton(_, phis):
                pa, pb = phis

                def dloop(d, carry):
                    ga1, ga2, ca, gb1, gb2, cb = carry
                    ma = ma_v[d]
                    mb = mb_v[d]
                    s1a = pa + ma
                    s2a = pa - ma
                    s1b = pb + mb
                    s2b = pb - mb
                    ga1 = ga1 + jnp.maximum(s1a, 0.0)
                    ga2 = ga2 + jnp.maximum(s2a, 0.0)
                    gb1 = gb1 + jnp.maximum(s1b, 0.0)
                    gb2 = gb2 + jnp.maximum(s2b, 0.0)
                    ca = ca + jnp.where(s1a > 0.0, 1.0, 0.0) \
                            + jnp.where(s2a > 0.0, 1.0, 0.0)
                    cb = cb + jnp.where(s1b > 0.0, 1.0, 0.0) \
                            + jnp.where(s2b > 0.0, 1.0, 0.0)
                    return ga1, ga2, ca, gb1, gb2, cb

                ga1, ga2, ca, gb1, gb2, cb = lax.fori_loop(
                    0, D, dloop, (zero, zero, zero, zero, zero, zero), unroll=2)
                ca = jnp.maximum(ca, 1.0)
                cb = jnp.maximum(cb, 1.0)
                pa = pa - (ga1 + ga2 - GAMMA) / ca
                pb = pb - (gb1 + gb2 - GAMMA) / cb
                return pa, pb

            pa, pb = lax.fori_loop(0, NEWTON_ITERS, newton, (phi0, phi0))
            out_v[p] = jnp.maximum(pa - pb, 0.0)
            return 0

        lax.fori_loop(0, PPW, pixel_body, 0)
        pltpu.sync_copy(out_v, out_hbm.at[pl.ds(wid * PPW, PPW)])

    return k(xb, wh)


def kernel(inputs, kernel):
    xpad = jnp.pad(inputs, ((0, 0), (1, 1), (1, 1), (0, 0)))
    xb = jnp.broadcast_to(xpad[..., None], xpad.shape + (FILTERS,))
    wh = kernel * 0.5
    out = _sc_spike_conv(xb, wh)
    return out.reshape(B, H, W, FILTERS)


# speedup vs baseline: 26.5510x; 1.2849x over previous
import functools

import jax
import jax.numpy as jnp
from jax import lax
from jax.experimental import pallas as pl
from jax.experimental.pallas import tpu as pltpu
from jax.experimental.pallas import tpu_sc as plsc

FILTERS = 16
KSIZE = 3
GAMMA = 1.0

B, H, W, C = 4, 32, 32, 16
D = C * KSIZE * KSIZE
S2 = 2 * D
NW = 32
PIX = B * H * W
PPW = PIX // NW
ROWS_PER_W = PPW // W
NEWTON_ITERS = 12


def _sc_spike_conv(xb, wh):
    mesh = plsc.VectorSubcoreMesh(core_axis_name="c", subcore_axis_name="s")

    @functools.partial(
        pl.kernel,
        out_type=jax.ShapeDtypeStruct((PIX, FILTERS), jnp.float32),
        mesh=mesh,
        compiler_params=pltpu.CompilerParams(use_tc_tiling_on_sc=False),
        scratch_types=[
            pltpu.VMEM((ROWS_PER_W + 2, W + 2, C, FILTERS), jnp.float32),
            pltpu.VMEM((D, FILTERS), jnp.float32),
            pltpu.VMEM((D, FILTERS), jnp.float32),
            pltpu.VMEM((D, FILTERS), jnp.float32),
            pltpu.VMEM((PPW, FILTERS), jnp.float32),
        ],
    )
    def k(xb_hbm, wh_hbm, out_hbm, slab_v, wh_v, ma_v, mb_v, out_v):
        wid = lax.axis_index("s") * 2 + lax.axis_index("c")
        img = wid // (H // ROWS_PER_W)
        row0 = (wid % (H // ROWS_PER_W)) * ROWS_PER_W
        pltpu.sync_copy(xb_hbm.at[img, pl.ds(row0, ROWS_PER_W + 2)], slab_v)
        pltpu.sync_copy(wh_hbm, wh_v)

        phi0 = jnp.full((FILTERS,), GAMMA / S2, dtype=jnp.float32)
        zero = jnp.zeros((FILTERS,), dtype=jnp.float32)

        def pixel_body(p, _):
            r = p // W
            col = p - r * W

            for dij in range(KSIZE * KSIZE):
                di, dj = dij // KSIZE, dij % KSIZE

                def build_c(c, _, di=di, dj=dj, dij=dij):
                    x = slab_v[r + di, col + dj, c]
                    wv = wh_v[dij * C + c]
                    ma_v[dij * C + c] = jnp.abs(x + wv)
                    mb_v[dij * C + c] = jnp.abs(x - wv)
                    return 0

                lax.fori_loop(0, C, build_c, 0, unroll=4)

            def newton(_, phis):
                pa, pb = phis

                def dloop(d, carry):
                    ga1, ga2, ca, gb1, gb2, cb = carry
                    ma = ma_v[d]
                    mb = mb_v[d]
                    s1a = pa + ma
                    s2a = pa - ma
                    s1b = pb + mb
                    s2b = pb - mb
                    ga1 = ga1 + jnp.maximum(s1a, 0.0)
                    ga2 = ga2 + jnp.maximum(s2a, 0.0)
                    gb1 = gb1 + jnp.maximum(s1b, 0.0)
                    gb2 = gb2 + jnp.maximum(s2b, 0.0)
                    ca = ca + jnp.where(s1a > 0.0, 1.0, 0.0) \
                            + jnp.where(s2a > 0.0, 1.0, 0.0)
                    cb = cb + jnp.where(s1b > 0.0, 1.0, 0.0) \
                            + jnp.where(s2b > 0.0, 1.0, 0.0)
                    return ga1, ga2, ca, gb1, gb2, cb

                ga1, ga2, ca, gb1, gb2, cb = lax.fori_loop(
                    0, D, dloop, (zero, zero, zero, zero, zero, zero), unroll=4)
                ca = jnp.maximum(ca, 1.0)
                cb = jnp.maximum(cb, 1.0)
                pa = pa - (ga1 + ga2 - GAMMA) / ca
                pb = pb - (gb1 + gb2 - GAMMA) / cb
                return pa, pb

            pa, pb = lax.fori_loop(0, NEWTON_ITERS, newton, (phi0, phi0))
            out_v[p] = jnp.maximum(pa - pb, 0.0)
            return 0

        lax.fori_loop(0, PPW, pixel_body, 0)
        pltpu.sync_copy(out_v, out_hbm.at[pl.ds(wid * PPW, PPW)])

    return k(xb, wh)


def kernel(inputs, kernel):
    xpad = jnp.pad(inputs, ((0, 0), (1, 1), (1, 1), (0, 0)))
    xb = jnp.broadcast_to(xpad[..., None], xpad.shape + (FILTERS,))
    wh = kernel * 0.5
    out = _sc_spike_conv(xb, wh)
    return out.reshape(B, H, W, FILTERS)


# speedup vs baseline: 28.4702x; 1.0723x over previous
import functools

import jax
import jax.numpy as jnp
from jax import lax
from jax.experimental import pallas as pl
from jax.experimental.pallas import tpu as pltpu
from jax.experimental.pallas import tpu_sc as plsc

FILTERS = 16
KSIZE = 3
GAMMA = 1.0

B, H, W, C = 4, 32, 32, 16
D = C * KSIZE * KSIZE
S2 = 2 * D
NW = 32
PIX = B * H * W
PPW = PIX // NW
ROWS_PER_W = PPW // W
NEWTON_ITERS = 12


def _sc_spike_conv(xb, wh, wmx):
    mesh = plsc.VectorSubcoreMesh(core_axis_name="c", subcore_axis_name="s")

    @functools.partial(
        pl.kernel,
        out_type=jax.ShapeDtypeStruct((PIX, FILTERS), jnp.float32),
        mesh=mesh,
        compiler_params=pltpu.CompilerParams(use_tc_tiling_on_sc=False),
        scratch_types=[
            pltpu.VMEM((ROWS_PER_W + 2, W + 2, C, FILTERS), jnp.float32),
            pltpu.VMEM((D, FILTERS), jnp.float32),
            pltpu.VMEM((D, FILTERS), jnp.float32),
            pltpu.VMEM((D, FILTERS), jnp.float32),
            pltpu.VMEM((D, FILTERS), jnp.float32),
            pltpu.VMEM((D, FILTERS), jnp.float32),
            pltpu.VMEM((D, FILTERS), jnp.float32),
            pltpu.VMEM((PPW, FILTERS), jnp.float32),
        ],
    )
    def k(xb_hbm, wh_hbm, wmx_hbm, out_hbm, slab_v, wh_v, wmax_v,
          ma_v, mb_v, bnda_v, bndb_v, out_v):
        wid = lax.axis_index("s") * 2 + lax.axis_index("c")
        img = wid // (H // ROWS_PER_W)
        row0 = (wid % (H // ROWS_PER_W)) * ROWS_PER_W
        pltpu.sync_copy(xb_hbm.at[img, pl.ds(row0, ROWS_PER_W + 2)], slab_v)
        pltpu.sync_copy(wh_hbm, wh_v)
        pltpu.sync_copy(wmx_hbm, wmax_v)

        phi0 = jnp.full((FILTERS,), GAMMA / S2, dtype=jnp.float32)
        zero = jnp.zeros((FILTERS,), dtype=jnp.float32)

        def pixel_body(p, _):
            r = p // W
            col = p - r * W

            for dij in range(KSIZE * KSIZE):
                di, dj = dij // KSIZE, dij % KSIZE

                def build_c(c, _, di=di, dj=dj, dij=dij):
                    x = slab_v[r + di, col + dj, c]
                    wv = wh_v[dij * C + c]
                    bnd = jnp.abs(x) + wmax_v[dij * C + c]
                    ma_v[dij * C + c] = jnp.abs(x + wv)
                    mb_v[dij * C + c] = jnp.abs(x - wv)
                    bnda_v[dij * C + c] = bnd
                    bndb_v[dij * C + c] = bnd
                    return 0

                lax.fori_loop(0, C, build_c, 0, unroll=4)

            def newton_pass(m_ref, phi, nd, unroll):
                def dl(d, carry):
                    g1, g2, c1, c2 = carry
                    m = m_ref[d]
                    s1 = phi + m
                    s2 = phi - m
                    return (g1 + jnp.maximum(s1, 0.0),
                            g2 + jnp.maximum(s2, 0.0),
                            c1 + jnp.where(s1 > 0.0, 1.0, 0.0),
                            c2 + jnp.where(s2 > 0.0, 1.0, 0.0))

                g1, g2, c1, c2 = lax.fori_loop(
                    0, nd, dl, (zero, zero, zero, zero), unroll=unroll)
                c = jnp.maximum(c1 + c2, 1.0)
                return phi - (g1 + g2 - GAMMA) / c

            def compact(m_ref, bnd_ref, phi, nd):
                mx = phi[0]
                for i in range(1, FILTERS):
                    mx = jnp.maximum(mx, phi[i])
                thr = -mx

                def comp(d, n):
                    bv = bnd_ref[d]
                    m_ref[n] = m_ref[d]
                    bnd_ref[n] = bv
                    return n + jnp.where(bv[0] > thr, 1, 0)

                return lax.fori_loop(0, nd, comp, 0)

            def solve(m_ref, bnd_ref):
                phi = newton_pass(m_ref, phi0, D, 4)
                phi = newton_pass(m_ref, phi, D, 4)
                n1 = compact(m_ref, bnd_ref, phi, D)
                phi = lax.fori_loop(
                    0, 2, lambda _, q: newton_pass(m_ref, q, n1, 1), phi)
                n2 = compact(m_ref, bnd_ref, phi, n1)
                phi = lax.fori_loop(
                    0, NEWTON_ITERS - 4,
                    lambda _, q: newton_pass(m_ref, q, n2, 1), phi)
                return phi

            pa = solve(ma_v, bnda_v)
            pb = solve(mb_v, bndb_v)
            out_v[p] = jnp.maximum(pa - pb, 0.0)
            return 0

        lax.fori_loop(0, PPW, pixel_body, 0)
        pltpu.sync_copy(out_v, out_hbm.at[pl.ds(wid * PPW, PPW)])

    return k(xb, wh, wmx)


def kernel(inputs, kernel):
    xpad = jnp.pad(inputs, ((0, 0), (1, 1), (1, 1), (0, 0)))
    xb = jnp.broadcast_to(xpad[..., None], xpad.shape + (FILTERS,))
    wh = kernel * 0.5
    wmx = jnp.broadcast_to(
        jnp.max(jnp.abs(wh), axis=1, keepdims=True), (D, FILTERS))
    out = _sc_spike_conv(xb, wh, wmx)
    return out.reshape(B, H, W, FILTERS)
